# baseline (speedup 1.0000x reference)
import functools

import jax
import jax.numpy as jnp
from jax.experimental import pallas as pl
from jax.experimental.pallas import tpu as pltpu

_TOPK = 128
_BQ = 128
_NEG = -1e9


def _float_keys(s):
    si = jax.lax.bitcast_convert_type(s, jnp.int32)
    return jnp.where(si < 0, si ^ jnp.int32(0x7FFFFFFF), si)


def _attn_body(q_ref, k_ref, v_ref, o_ref, s_ref):
    qb = pl.program_id(1)
    q = q_ref[0]
    bq, d = q.shape
    nt = qb + 1
    scale = jnp.float32(1.0) / jnp.sqrt(jnp.float32(d))
    tri = (jax.lax.broadcasted_iota(jnp.int32, (bq, bq), 1)
           <= jax.lax.broadcasted_iota(jnp.int32, (bq, bq), 0))

    def qk_body(j, m):
        kt = k_ref[0, pl.ds(j * bq, bq), :]
        st = jax.lax.dot_general(
            q, kt, (((1,), (1,)), ((), ())),
            preferred_element_type=jnp.float32,
            precision=jax.lax.Precision.DEFAULT) * scale
        st = jnp.where(jnp.logical_or(j < qb, tri), st, jnp.float32(_NEG))
        s_ref[:, pl.ds(j * bq, bq)] = st
        return jnp.maximum(m, jnp.max(st, axis=-1, keepdims=True))

    m = jax.lax.fori_loop(
        0, nt, qk_body, jnp.full((bq, 1), _NEG, jnp.float32))

    lo = m - jnp.float32(25.0)
    hi = m

    def bisect(_, carry):
        lo, hi = carry
        mid = jnp.float32(0.5) * (lo + hi)

        def cbody(j, acc):
            st = s_ref[:, pl.ds(j * bq, bq)]
            return acc + jnp.where(st >= mid, jnp.float32(1.0),
                                   jnp.float32(0.0))

        acc = jax.lax.fori_loop(
            0, nt, cbody, jnp.zeros((bq, bq), jnp.float32))
        cnt = jnp.sum(acc, axis=-1, keepdims=True)
        ge = cnt >= _TOPK
        return jnp.where(ge, mid, lo), jnp.where(ge, hi, mid)

    lo, hi = jax.lax.fori_loop(0, 20, bisect, (lo, hi))

    def pv_body(j, carry):
        o_acc, den_part = carry
        st = s_ref[:, pl.ds(j * bq, bq)]
        pt = jnp.where(st >= lo, jnp.exp(st - m), jnp.float32(0.0))
        vt = v_ref[0, pl.ds(j * bq, bq), :]
        o_acc = o_acc + jax.lax.dot_general(
            pt, vt, (((1,), (0,)), ((), ())),
            preferred_element_type=jnp.float32,
            precision=jax.lax.Precision.DEFAULT)
        return o_acc, den_part + pt

    o_acc, den_part = jax.lax.fori_loop(
        0, nt, pv_body,
        (jnp.zeros((bq, d), jnp.float32), jnp.zeros((bq, bq), jnp.float32)))
    den = jnp.sum(den_part, axis=-1, keepdims=True)
    o_ref[0] = o_acc / den


def _build_call(bh, s_len, d, interpret=False):
    grid = (bh, s_len // _BQ)
    return pl.pallas_call(
        _attn_body,
        grid=grid,
        in_specs=[
            pl.BlockSpec((1, _BQ, d), lambda b, i: (b, i, 0)),
            pl.BlockSpec((1, s_len, d), lambda b, i: (b, 0, 0)),
            pl.BlockSpec((1, s_len, d), lambda b, i: (b, 0, 0)),
        ],
        out_specs=pl.BlockSpec((1, _BQ, d), lambda b, i: (b, i, 0)),
        out_shape=jax.ShapeDtypeStruct((bh, s_len, d), jnp.float32),
        scratch_shapes=[pltpu.VMEM((_BQ, s_len), jnp.float32)],
        compiler_params=pltpu.CompilerParams(
            dimension_semantics=("parallel", "arbitrary")),
        interpret=interpret,
    )


@jax.jit
def kernel(q, k, v):
    b, h, s_len, d = q.shape
    qf = q.reshape(b * h, s_len, d)
    kf = k.reshape(b * h, s_len, d)
    vf = v.reshape(b * h, s_len, d)
    out = _build_call(b * h, s_len, d)(qf, kf, vf)
    return out.reshape(b, h, s_len, d)


# speedup vs baseline: 2.1195x; 2.1195x over previous
import functools

import jax
import jax.numpy as jnp
from jax.experimental import pallas as pl
from jax.experimental.pallas import tpu as pltpu

_TOPK = 128
_BQ = 128
_NEG = -1e9
_BISECT_ITERS = 20


def _make_body(qb, nq):
    def body(q_ref, k_ref, v_ref, o_ref):
        q = q_ref[0]
        k = k_ref[0]
        v = v_ref[0]
        bq, d = q.shape
        w = k.shape[0]
        scale = jnp.float32(1.0) / jnp.sqrt(jnp.float32(d))

        s = jax.lax.dot_general(
            q, k, (((1,), (1,)), ((), ())),
            preferred_element_type=jnp.float32,
            precision=jax.lax.Precision.DEFAULT) * scale

        row = qb * bq + jax.lax.broadcasted_iota(jnp.int32, (bq, w), 0)
        col = jax.lax.broadcasted_iota(jnp.int32, (bq, w), 1)
        s = jnp.where(col <= row, s, jnp.float32(_NEG))

        m = jnp.max(s, axis=-1, keepdims=True)

        if qb == 0:
            p = jnp.exp(s - m)
        else:
            lo = m - jnp.float32(25.0)
            hi = m

            def bisect(_, carry):
                lo, hi = carry
                mid = jnp.float32(0.5) * (lo + hi)
                cnt = jnp.sum(
                    jnp.where(s >= mid, jnp.float32(1.0), jnp.float32(0.0)),
                    axis=-1, keepdims=True)
                ge = cnt >= _TOPK
                return jnp.where(ge, mid, lo), jnp.where(ge, hi, mid)

            lo, hi = jax.lax.fori_loop(0, _BISECT_ITERS, bisect, (lo, hi))
            p = jnp.where(s >= lo, jnp.exp(s - m), jnp.float32(0.0))

        den = jnp.sum(p, axis=-1, keepdims=True)
        o = jax.lax.dot_general(
            p, v, (((1,), (0,)), ((), ())),
            preferred_element_type=jnp.float32,
            precision=jax.lax.Precision.DEFAULT)
        o_ref[0] = o / den

    return body


def _block_call(qb, nq, bh, s_len, d, interpret=False):
    w = (qb + 1) * _BQ
    return pl.pallas_call(
        _make_body(qb, nq),
        grid=(bh,),
        in_specs=[
            pl.BlockSpec((1, _BQ, d), lambda b: (b, qb, 0)),
            pl.BlockSpec((1, w, d), lambda b: (b, 0, 0)),
            pl.BlockSpec((1, w, d), lambda b: (b, 0, 0)),
        ],
        out_specs=pl.BlockSpec((1, _BQ, d), lambda b: (b, 0, 0)),
        out_shape=jax.ShapeDtypeStruct((bh, _BQ, d), jnp.float32),
        compiler_params=pltpu.CompilerParams(
            dimension_semantics=("parallel",)),
        interpret=interpret,
    )


def _run(q, k, v, interpret=False):
    b, h, s_len, d = q.shape
    bh = b * h
    nq = s_len // _BQ
    qf = q.reshape(bh, s_len, d)
    kf = k.reshape(bh, s_len, d)
    vf = v.reshape(bh, s_len, d)
    slabs = [
        _block_call(qb, nq, bh, s_len, d, interpret)(qf, kf, vf)
        for qb in range(nq)
    ]
    return jnp.concatenate(slabs, axis=1).reshape(b, h, s_len, d)


@jax.jit
def kernel(q, k, v):
    return _run(q, k, v)
